# baseline (speedup 1.0000x reference)
import functools

import jax
import jax.numpy as jnp
from jax import lax
from jax.experimental import pallas as pl
from jax.experimental.pallas import tpu as pltpu
from jax.experimental.pallas import tpu_sc as plsc

N_NODES = 10000
N_EDGES = 320000
D = 128

NC = 2
NS = 16
NW = NC * NS
E_PER_W = N_EDGES // NW
CHUNK = 100
GROUP = 6
N_GROUPS = 17
E_PAD_W = N_GROUPS * GROUP * CHUNK
N_PAD = 10240
ROWS_PER_S = N_PAD // NS
RZ = 80


def _sc_aggregate(feat, edge_il):
    mesh = plsc.VectorSubcoreMesh(core_axis_name="c", subcore_axis_name="s")

    @functools.partial(
        pl.kernel,
        mesh=mesh,
        out_type=jax.ShapeDtypeStruct((NC * N_PAD, D), jnp.float32),
        scratch_types=[
            pltpu.VMEM((2, GROUP, 2, CHUNK), jnp.int32),
            pltpu.VMEM((CHUNK, D), jnp.float32),
            pltpu.VMEM((CHUNK, D), jnp.float32),
            pltpu.VMEM((CHUNK, D), jnp.float32),
            pltpu.VMEM_SHARED((N_PAD, D), jnp.float32),
            pltpu.SemaphoreType.DMA,
            pltpu.SemaphoreType.DMA,
            pltpu.SemaphoreType.DMA,
            pltpu.SemaphoreType.DMA,
        ],
    )
    def agg_kernel(feat_hbm, idx_hbm, out_hbm,
                   idx_v, r0, r1, r2, acc, s0, s1, s2, isem):
        c = lax.axis_index("c")
        s = lax.axis_index("s")
        wid = c * NS + s
        bufs = (r0, r1, r2)
        sems = (s0, s1, s2)

        pltpu.sync_copy(idx_hbm.at[wid, 0], idx_v.at[0])
        zero16 = jnp.zeros((16,), jnp.float32)

        def zero_row(i, carry):
            for j in range(D // 16):
                r0[i, pl.ds(j * 16, 16)] = zero16
            return carry

        lax.fori_loop(0, RZ, zero_row, 0)
        zsrc = r0.at[pl.ds(0, RZ)]
        nz = ROWS_PER_S // RZ
        for k in range(nz):
            pltpu.async_copy(
                zsrc, acc.at[pl.ds(s * ROWS_PER_S + k * RZ, RZ)], isem)
        for k in range(nz):
            pltpu.make_async_copy(
                zsrc, acc.at[pl.ds(s * ROWS_PER_S, RZ)], isem).wait()

        for j in range(3):
            pltpu.async_copy(
                feat_hbm.at[idx_v.at[0, j, 0]], bufs[j], sems[j])
        plsc.subcore_barrier()

        def group_body(g, carry):
            p = lax.rem(g, 2)
            pn = 1 - p
            @pl.when(g + 1 < N_GROUPS)
            def _():
                pltpu.async_copy(idx_hbm.at[wid, g + 1], idx_v.at[pn], isem)

            for j in range(GROUP):
                buf, sem = bufs[j % 3], sems[j % 3]
                pltpu.make_async_copy(
                    feat_hbm.at[idx_v.at[0, 0, 0]], buf, sem).wait()
                pltpu.sync_copy(buf, acc.at[idx_v.at[p, j, 1]], add=True)
                if j < 3:
                    pltpu.async_copy(
                        feat_hbm.at[idx_v.at[p, j + 3, 0]], buf, sem)
                else:
                    @pl.when(g + 1 < N_GROUPS)
                    def _(j=j):
                        if j == 3:
                            pltpu.make_async_copy(
                                idx_hbm.at[wid, 0], idx_v.at[pn],
                                isem).wait()
                        pltpu.async_copy(
                            feat_hbm.at[idx_v.at[pn, j - 3, 0]], buf, sem)
            return carry

        lax.fori_loop(0, N_GROUPS, group_body, 0)
        plsc.subcore_barrier()

        obufs = (r0.at[pl.ds(0, RZ)], r1.at[pl.ds(0, RZ)])
        for k in range(ROWS_PER_S // RZ):
            ob, sem = obufs[k % 2], sems[k % 2]
            if k >= 2:
                pltpu.make_async_copy(
                    ob, out_hbm.at[pl.ds(c * N_PAD, RZ)], sem).wait()
            rr = s * ROWS_PER_S + k * RZ
            pltpu.sync_copy(acc.at[pl.ds(rr, RZ)], ob)
            pltpu.async_copy(ob, out_hbm.at[pl.ds(c * N_PAD + rr, RZ)], sem)
        for k in range(2):
            pltpu.make_async_copy(
                obufs[k], out_hbm.at[pl.ds(c * N_PAD, RZ)], sems[k]).wait()

    return agg_kernel(feat, edge_il)


def _tc_body(p_ref, f_ref, wn_ref, ws_ref, b_ref, o_ref):
    agg = p_ref[0] + p_ref[1]
    o_ref[...] = (
        jnp.dot(agg, wn_ref[...], preferred_element_type=jnp.float32)
        + jnp.dot(f_ref[...], ws_ref[...], preferred_element_type=jnp.float32)
        + b_ref[...]
    )


def kernel(feat, edge_index, W_neigh, b_neigh, W_self):
    ei = edge_index.astype(jnp.int32).reshape(2, NW, E_PER_W)
    pad = E_PAD_W - E_PER_W
    srcw = jnp.pad(ei[0], ((0, 0), (0, pad)))
    dstw = jnp.pad(ei[1], ((0, 0), (0, pad)), constant_values=N_PAD - 1)
    edge_il = jnp.stack(
        [srcw.reshape(NW, N_GROUPS, GROUP, CHUNK),
         dstw.reshape(NW, N_GROUPS, GROUP, CHUNK)], axis=3)

    partials = _sc_aggregate(feat, edge_il).reshape(NC, N_PAD, D)

    B = 1000
    out = pl.pallas_call(
        _tc_body,
        grid=(N_NODES // B,),
        in_specs=[
            pl.BlockSpec((NC, B, D), lambda i: (0, i, 0)),
            pl.BlockSpec((B, D), lambda i: (i, 0)),
            pl.BlockSpec((D, D), lambda i: (0, 0)),
            pl.BlockSpec((D, D), lambda i: (0, 0)),
            pl.BlockSpec((1, D), lambda i: (0, 0)),
        ],
        out_specs=pl.BlockSpec((B, D), lambda i: (i, 0)),
        out_shape=jax.ShapeDtypeStruct((N_NODES, D), jnp.float32),
    )(partials, feat, W_neigh, W_self, b_neigh.reshape(1, D))
    return out


# speedup vs baseline: 2.7685x; 2.7685x over previous
import functools

import jax
import jax.numpy as jnp
from jax import lax
from jax.experimental import pallas as pl
from jax.experimental.pallas import tpu as pltpu
from jax.experimental.pallas import tpu_sc as plsc

N_NODES = 10000
N_EDGES = 320000
D = 128

NC = 2
NS = 16
NW = NC * NS
E_PER_W = N_EDGES // NW
CHUNK = 125
N_STEPS = E_PER_W // CHUNK
N_QUADS = N_STEPS // 4
N_DUOS = N_QUADS // 2
N_PAD = 10240
ROWS_PER_S = N_PAD // NS
RZ = 80


def _sc_aggregate(feat, edge_il):
    mesh = plsc.VectorSubcoreMesh(core_axis_name="c", subcore_axis_name="s")

    @functools.partial(
        pl.kernel,
        mesh=mesh,
        out_type=jax.ShapeDtypeStruct((NC * N_PAD, D), jnp.float32),
        scratch_types=[
            pltpu.VMEM((4, 2, CHUNK), jnp.int32),
            pltpu.VMEM((4, 2, CHUNK), jnp.int32),
            pltpu.VMEM((CHUNK, D), jnp.float32),
            pltpu.VMEM((CHUNK, D), jnp.float32),
            pltpu.VMEM_SHARED((N_PAD, D), jnp.float32),
            pltpu.SemaphoreType.DMA,
            pltpu.SemaphoreType.DMA,
            pltpu.SemaphoreType.DMA,
        ],
    )
    def agg_kernel(feat_hbm, idx_hbm, out_hbm,
                   idx_p, idx_q, rows_a, rows_b, acc, sa, sb, isem):
        c = lax.axis_index("c")
        s = lax.axis_index("s")
        wid = c * NS + s

        pltpu.sync_copy(idx_hbm.at[wid, 0], idx_p)
        zero16 = jnp.zeros((16,), jnp.float32)

        def zero_row(i, carry):
            for j in range(D // 16):
                rows_a[i, pl.ds(j * 16, 16)] = zero16
            return carry

        lax.fori_loop(0, RZ, zero_row, 0)
        zsrc = rows_a.at[pl.ds(0, RZ)]
        nz = ROWS_PER_S // RZ
        for k in range(nz):
            pltpu.async_copy(
                zsrc, acc.at[pl.ds(s * ROWS_PER_S + k * RZ, RZ)], isem)
        for k in range(nz):
            pltpu.make_async_copy(
                zsrc, acc.at[pl.ds(s * ROWS_PER_S, RZ)], isem).wait()

        pltpu.async_copy(feat_hbm.at[idx_p.at[0, 0]], rows_a, sa)
        pltpu.async_copy(feat_hbm.at[idx_p.at[1, 0]], rows_b, sb)
        plsc.subcore_barrier()

        bufs = (rows_a, rows_b)
        sems = (sa, sb)

        def run_quad(idx_cur, idx_nxt, have_next):
            for j in range(4):
                buf, sem = bufs[j % 2], sems[j % 2]
                pltpu.make_async_copy(
                    feat_hbm.at[idx_cur.at[0, 0]], buf, sem).wait()
                pltpu.sync_copy(buf, acc.at[idx_cur.at[j, 1]], add=True)
                if j < 2:
                    pltpu.async_copy(
                        feat_hbm.at[idx_cur.at[j + 2, 0]], buf, sem)
                else:
                    @pl.when(have_next)
                    def _(j=j):
                        if j == 2:
                            pltpu.make_async_copy(
                                idx_hbm.at[wid, 0], idx_nxt, isem).wait()
                        pltpu.async_copy(
                            feat_hbm.at[idx_nxt.at[j - 2, 0]], buf, sem)

        def duo(k, carry):
            q0 = 2 * k
            pltpu.async_copy(idx_hbm.at[wid, q0 + 1], idx_q, isem)
            run_quad(idx_p, idx_q, q0 + 1 < N_QUADS)

            @pl.when(q0 + 2 < N_QUADS)
            def _():
                pltpu.async_copy(idx_hbm.at[wid, q0 + 2], idx_p, isem)

            run_quad(idx_q, idx_p, q0 + 2 < N_QUADS)
            return carry

        lax.fori_loop(0, N_DUOS, duo, 0)
        plsc.subcore_barrier()

        obufs = (rows_a.at[pl.ds(0, RZ)], rows_b.at[pl.ds(0, RZ)])
        for k in range(ROWS_PER_S // RZ):
            ob, sem = obufs[k % 2], sems[k % 2]
            if k >= 2:
                pltpu.make_async_copy(
                    ob, out_hbm.at[pl.ds(c * N_PAD, RZ)], sem).wait()
            rr = s * ROWS_PER_S + k * RZ
            pltpu.sync_copy(acc.at[pl.ds(rr, RZ)], ob)
            pltpu.async_copy(ob, out_hbm.at[pl.ds(c * N_PAD + rr, RZ)], sem)
        for k in range(2):
            pltpu.make_async_copy(
                obufs[k], out_hbm.at[pl.ds(c * N_PAD, RZ)], sems[k]).wait()

    return agg_kernel(feat, edge_il)


def _tc_body(p_ref, f_ref, wn_ref, ws_ref, b_ref, o_ref):
    agg = p_ref[0] + p_ref[1]
    o_ref[...] = (
        jnp.dot(agg, wn_ref[...], preferred_element_type=jnp.float32)
        + jnp.dot(f_ref[...], ws_ref[...], preferred_element_type=jnp.float32)
        + b_ref[...]
    )


def kernel(feat, edge_index, W_neigh, b_neigh, W_self):
    edge_il = edge_index.astype(jnp.int32) \
        .reshape(2, NW, N_QUADS, 4, CHUNK).transpose(1, 2, 3, 0, 4)

    partials = _sc_aggregate(feat, edge_il).reshape(NC, N_PAD, D)

    B = 1000
    out = pl.pallas_call(
        _tc_body,
        grid=(N_NODES // B,),
        in_specs=[
            pl.BlockSpec((NC, B, D), lambda i: (0, i, 0)),
            pl.BlockSpec((B, D), lambda i: (i, 0)),
            pl.BlockSpec((D, D), lambda i: (0, 0)),
            pl.BlockSpec((D, D), lambda i: (0, 0)),
            pl.BlockSpec((1, D), lambda i: (0, 0)),
        ],
        out_specs=pl.BlockSpec((B, D), lambda i: (i, 0)),
        out_shape=jax.ShapeDtypeStruct((N_NODES, D), jnp.float32),
    )(partials, feat, W_neigh, W_self, b_neigh.reshape(1, D))
    return out
